# Initial kernel scaffold; baseline (speedup 1.0000x reference)
import jax
import jax.numpy as jnp
from jax.experimental import pallas as pl


def kernel(x, edge_index, edge_attr, rand_feat, params):
    raise NotImplementedError("write your pallas kernel here")


# speedup vs baseline: 15.4869x; 15.4869x over previous
import functools

import jax
import jax.numpy as jnp
from jax import lax
from jax.experimental import pallas as pl
from jax.experimental.pallas import tpu as pltpu
from jax.experimental.pallas import tpu_sc as plsc

H = 64
N = 50000
E = 800000
NP = 51200
EP = 819200
NC = 2
NS = 16
NW = NC * NS
EW = EP // NW
B = 1600
NB = EW // B
CH = 8
CR = NP // CH
ET = EP // NS
B3 = 3200
NB3 = ET // B3
RB = 64
CAP = B3 + RB

_mesh = plsc.VectorSubcoreMesh(core_axis_name="c", subcore_axis_name="s",
                               num_cores=NC, num_subcores=NS)
_sc_params = pltpu.CompilerParams(use_tc_tiling_on_sc=False,
                                  needs_layout_passes=False)
f32 = jnp.float32
i32 = jnp.int32


def _tc_node_mlp(xp, uc8, ut8, b1c, b1t, w2c, w2t, b2c, b2t):
    BN = 1024

    def body(x_ref, uc_ref, ut_ref, b1c_ref, b1t_ref, w2c_ref, w2t_ref,
             b2c_ref, b2t_ref, o_ref):
        xb = x_ref[...]
        m = xb[:, 0:1]
        hc = jnp.maximum(jnp.dot(xb, uc_ref[...],
                                 preferred_element_type=f32) + b1c_ref[...], 0.0)
        ht = jnp.maximum(jnp.dot(xb, ut_ref[...],
                                 preferred_element_type=f32) + b1t_ref[...], 0.0)
        mc = jnp.dot(hc, w2c_ref[...], preferred_element_type=f32) + b2c_ref[...]
        mt = jnp.dot(ht, w2t_ref[...], preferred_element_type=f32) + b2t_ref[...]
        o_ref[...] = mc * (1.0 - m) + mt * m

    full = lambda s: pl.BlockSpec(s, lambda i: (0, 0))
    return pl.pallas_call(
        body,
        grid=(NP // BN,),
        in_specs=[pl.BlockSpec((BN, 8), lambda i: (i, 0)),
                  full((8, 32)), full((8, 32)), full((1, 32)), full((1, 32)),
                  full((32, H)), full((32, H)), full((1, H)), full((1, H))],
        out_specs=pl.BlockSpec((BN, H), lambda i: (i, 0)),
        out_shape=jax.ShapeDtypeStruct((NP, H), f32),
    )(xp, uc8, ut8, b1c, b1t, w2c, w2t, b2c, b2t)


def _tc_edge_terms(eap, ve8, b1e, me8, ce8):
    BE = 2048

    def body(e_ref, ve_ref, b1e_ref, me_ref, ce_ref, o_ref):
        hid = jnp.maximum(jnp.dot(e_ref[...], ve_ref[...],
                                  preferred_element_type=f32) + b1e_ref[...], 0.0)
        o_ref[...] = jnp.dot(hid, me_ref[...],
                             preferred_element_type=f32) + ce_ref[...]

    full = lambda s: pl.BlockSpec(s, lambda i: (0, 0))
    return pl.pallas_call(
        body,
        grid=(EP // BE,),
        in_specs=[pl.BlockSpec((BE, 8), lambda i: (i, 0)),
                  full((8, 32)), full((1, 32)), full((32, 8)), full((1, 8))],
        out_specs=pl.BlockSpec((BE, 8), lambda i: (i, 0)),
        out_shape=jax.ShapeDtypeStruct((EP, 8), f32),
    )(eap, ve8, b1e, me8, ce8)


def _tc_proj(X, scale8, bias_row, W):
    BN = 512
    D = X.shape[1]
    K = W.shape[1]

    def body(x_ref, s_ref, b_ref, w_ref, o_ref):
        x = x_ref[...]
        s = s_ref[...]
        sx = jnp.concatenate(
            [jnp.broadcast_to(s[:, h:h + 1], (BN, H)) for h in range(D // H)],
            axis=1)
        o_ref[...] = jnp.dot(x * sx + b_ref[...], w_ref[...],
                             preferred_element_type=f32)

    return pl.pallas_call(
        body,
        grid=(NP // BN,),
        in_specs=[pl.BlockSpec((BN, D), lambda i: (i, 0)),
                  pl.BlockSpec((BN, 8), lambda i: (i, 0)),
                  pl.BlockSpec((1, D), lambda i: (0, 0)),
                  pl.BlockSpec((D, K), lambda i: (0, 0))],
        out_specs=pl.BlockSpec((BN, K), lambda i: (i, 0)),
        out_shape=jax.ShapeDtypeStruct((NP, K), f32),
    )(X, scale8, bias_row, W)


def _tc_reduce_max(mparts):
    GR = 10240
    nh = len(mparts)

    def body(*refs):
        ins, outs = refs[:nh], refs[nh:]
        for k in range(nh):
            outs[k][...] = jnp.max(ins[k][...], axis=0)

    return pl.pallas_call(
        body,
        grid=(NP // GR,),
        in_specs=[pl.BlockSpec((NW, GR), lambda i: (0, i)) for _ in range(nh)],
        out_specs=[pl.BlockSpec((GR,), lambda i: (i,)) for _ in range(nh)],
        out_shape=[jax.ShapeDtypeStruct((NP,), f32) for _ in range(nh)],
    )(*mparts)


def _tc_reduce_recip(denparts):
    GR = 10240
    nh = len(denparts)

    def body(*refs):
        ins, outs = refs[:nh], refs[nh:]
        for k in range(nh):
            d = ins[k][...]
            outs[k][...] = 1.0 / jnp.maximum(d[0, :] + d[1, :], 1e-16)

    return pl.pallas_call(
        body,
        grid=(NP // GR,),
        in_specs=[pl.BlockSpec((NC, GR), lambda i: (0, i)) for _ in range(nh)],
        out_specs=[pl.BlockSpec((GR,), lambda i: (i,)) for _ in range(nh)],
        out_shape=[jax.ShapeDtypeStruct((NP,), f32) for _ in range(nh)],
    )(*denparts)


def _tc_final(h3part, recip3, x0, bc3):
    def body(h_ref, r_ref, x_ref, b_ref, o_ref):
        h = h_ref[...]
        s = (h[0, :] + h[1, :]) * r_ref[...]
        o_ref[...] = (s[:N] + b_ref[...]) * x_ref[...]

    return pl.pallas_call(
        body,
        in_specs=[pl.BlockSpec((NC, NP), lambda: (0, 0)),
                  pl.BlockSpec((NP,), lambda: (0,)),
                  pl.BlockSpec((N,), lambda: (0,)),
                  pl.BlockSpec((1,), lambda: (0,))],
        out_specs=pl.BlockSpec((N,), lambda: (0,)),
        out_shape=jax.ShapeDtypeStruct((N,), f32),
    )(h3part, recip3, x0, bc3)


def _sc_logits_max(nh, src, dst, ales, asns, adns, neginf):
    out_type = tuple([jax.ShapeDtypeStruct((EP,), f32) for _ in range(nh)]
                     + [jax.ShapeDtypeStruct((NW, NP), f32) for _ in range(nh)])
    scratch = [pltpu.VMEM((NP,), f32),
               pltpu.VMEM_SHARED((NP,), f32),
               pltpu.VMEM_SHARED((NP,), f32),
               pltpu.VMEM((B,), i32),
               pltpu.VMEM((B,), i32),
               pltpu.VMEM((B,), f32),
               pltpu.VMEM((B,), f32),
               pltpu.VMEM((B,), f32),
               pltpu.VMEM((B,), f32),
               pltpu.SemaphoreType.DMA,
               pltpu.SemaphoreType.DMA]

    @functools.partial(pl.kernel, out_type=out_type, mesh=_mesh,
                       scratch_types=scratch,
                       compiler_params=_sc_params)
    def k(*refs):
        (src_h, dst_h), rest = refs[:2], refs[2:]
        ale_h, rest = rest[:nh], rest[nh:]
        asn_h, rest = rest[:nh], rest[nh:]
        adn_h, rest = rest[:nh], rest[nh:]
        (neg_h,), rest = rest[:1], rest[1:]
        al_o, rest = rest[:nh], rest[nh:]
        mp_o, rest = rest[:nh], rest[nh:]
        (mpriv, as_sp, ad_sp, srcv, dstv, alev, asv, adv, alv, sem, sem2) = rest
        cid = lax.axis_index("c")
        sid = lax.axis_index("s")
        wid = sid * NC + cid
        ebase = wid * EW
        for h in range(nh):
            @pl.when(sid == 0)
            def _():
                pltpu.sync_copy(asn_h[h], as_sp)
                pltpu.sync_copy(adn_h[h], ad_sp)
            pltpu.sync_copy(neg_h, mpriv)
            plsc.subcore_barrier()

            def batch(b, carry):
                off = ebase + b * B
                pltpu.sync_copy(src_h.at[pl.ds(off, B)], srcv)
                pltpu.sync_copy(dst_h.at[pl.ds(off, B)], dstv)
                pltpu.sync_copy(ale_h[h].at[pl.ds(off, B)], alev)
                pltpu.async_copy(as_sp.at[srcv], asv, sem).wait()
                pltpu.async_copy(ad_sp.at[dstv], adv, sem2).wait()

                def grp(g, c2):
                    sl = pl.ds(g * 16, 16)
                    a = asv[sl] + adv[sl] + alev[sl]
                    a = jnp.where(a >= 0.0, a, 0.2 * a)
                    alv[sl] = a
                    idx = dstv[sl]
                    m = plsc.load_gather(mpriv, [idx])
                    mask = a > m

                    def wbody(mk):
                        plsc.store_scatter(mpriv, [idx], a, mask=mk)
                        m2 = plsc.load_gather(mpriv, [idx])
                        return jnp.logical_and(mk, a > m2)

                    lax.while_loop(jnp.any, wbody, mask)
                    return c2

                lax.fori_loop(0, B // 16, grp, 0)
                pltpu.sync_copy(alv, al_o[h].at[pl.ds(off, B)])
                return carry

            lax.fori_loop(0, NB, batch, 0)
            pltpu.sync_copy(mpriv, mp_o[h].at[wid])
            plsc.subcore_barrier()

    return k(src, dst, *ales, *asns, *adns, neginf)


def _sc_exp_den(nh, dst, als, amaxs, zeros_n):
    out_type = tuple([jax.ShapeDtypeStruct((EP,), f32) for _ in range(nh)]
                     + [jax.ShapeDtypeStruct((NC, NP), f32) for _ in range(nh)])
    scratch = [pltpu.VMEM_SHARED((NP,), f32),
               pltpu.VMEM_SHARED((NP,), f32),
               pltpu.VMEM((B,), i32),
               pltpu.VMEM((B,), f32),
               pltpu.VMEM((B,), f32),
               pltpu.VMEM((B,), f32),
               pltpu.SemaphoreType.DMA]

    @functools.partial(pl.kernel, out_type=out_type, mesh=_mesh,
                       scratch_types=scratch,
                       compiler_params=_sc_params)
    def k(*refs):
        (dst_h,), rest = refs[:1], refs[1:]
        al_h, rest = rest[:nh], rest[nh:]
        am_h, rest = rest[:nh], rest[nh:]
        (z_h,), rest = rest[:1], rest[1:]
        p_o, rest = rest[:nh], rest[nh:]
        den_o, rest = rest[:nh], rest[nh:]
        (am_sp, den_sp, dstv, alv, amv, pv, sem) = rest
        cid = lax.axis_index("c")
        sid = lax.axis_index("s")
        wid = sid * NC + cid
        ebase = wid * EW
        for h in range(nh):
            @pl.when(sid == 0)
            def _():
                pltpu.sync_copy(am_h[h], am_sp)
                pltpu.sync_copy(z_h, den_sp)
            plsc.subcore_barrier()

            def batch(b, carry):
                off = ebase + b * B
                pltpu.sync_copy(dst_h.at[pl.ds(off, B)], dstv)
                pltpu.sync_copy(al_h[h].at[pl.ds(off, B)], alv)
                pltpu.async_copy(am_sp.at[dstv], amv, sem).wait()

                def grp(g, c2):
                    sl = pl.ds(g * 16, 16)
                    pv[sl] = jnp.exp(alv[sl] - amv[sl])
                    return c2

                lax.fori_loop(0, B // 16, grp, 0)
                pltpu.sync_copy(pv, p_o[h].at[pl.ds(off, B)])
                pltpu.sync_copy(pv, den_sp.at[dstv], add=True)
                return carry

            lax.fori_loop(0, NB, batch, 0)
            plsc.subcore_barrier()
            @pl.when(sid == 0)
            def _():
                pltpu.sync_copy(den_sp, den_o[h].at[cid])
            plsc.subcore_barrier()

    return k(dst, *als, *amaxs, zeros_n)


def _sc_aggregate(src, dst, p0, p1, p2, xs, zeros_chunk):
    out_type = jax.ShapeDtypeStruct((NP, 192), f32)
    scratch = [pltpu.VMEM_SHARED((CR, 192), f32),
               pltpu.VMEM((B3,), i32),
               pltpu.VMEM((B3,), i32),
               pltpu.VMEM((B3,), f32),
               pltpu.VMEM((B3,), f32),
               pltpu.VMEM((B3,), f32),
               pltpu.VMEM((CAP,), i32),
               pltpu.VMEM((CAP,), i32),
               pltpu.VMEM((CAP,), f32),
               pltpu.VMEM((CAP,), f32),
               pltpu.VMEM((CAP,), f32),
               pltpu.VMEM((RB,), i32),
               pltpu.VMEM((RB,), i32),
               pltpu.VMEM((RB, 192), f32),
               pltpu.SemaphoreType.DMA,
               pltpu.SemaphoreType.DMA]

    @functools.partial(pl.kernel, out_type=out_type, mesh=_mesh,
                       scratch_types=scratch,
                       compiler_params=_sc_params)
    def k(src_h, dst_h, p0_h, p1_h, p2_h, xs_h, zc_h, out_h,
          acc_sp, srcv, dstv, p0v, p1v, p2v,
          csrc, cloff, cp0, cp1, cp2, sblk, oblk, rowbuf, sem, sem2):
        cid = lax.axis_index("c")
        sid = lax.axis_index("s")
        pvs = (p0v, p1v, p2v)
        cps = (cp0, cp1, cp2)
        for cpass in range(CH // NC):
            chunk = cpass * NC + cid
            lo = chunk * CR
            @pl.when(sid == 0)
            def _():
                pltpu.sync_copy(zc_h, acc_sp)
            plsc.subcore_barrier()
            ebase = sid * ET

            def batch(b, carry):
                off = ebase + b * B3
                pltpu.sync_copy(src_h.at[pl.ds(off, B3)], srcv)
                pltpu.sync_copy(dst_h.at[pl.ds(off, B3)], dstv)
                pltpu.sync_copy(p0_h.at[pl.ds(off, B3)], p0v)
                pltpu.sync_copy(p1_h.at[pl.ds(off, B3)], p1v)
                pltpu.sync_copy(p2_h.at[pl.ds(off, B3)], p2v)

                def grp(g, cnt):
                    sl = pl.ds(g * 16, 16)
                    d16 = dstv[sl]
                    inb = jnp.logical_and(d16 >= lo, d16 < lo + CR)
                    plsc.store_compressed(csrc.at[pl.ds(cnt, 16)], srcv[sl],
                                          mask=inb)
                    plsc.store_compressed(cloff.at[pl.ds(cnt, 16)], d16 - lo,
                                          mask=inb)
                    for hh in range(3):
                        plsc.store_compressed(cps[hh].at[pl.ds(cnt, 16)],
                                              pvs[hh][sl], mask=inb)
                    npop = plsc.all_reduce_population_count(inb)
                    return cnt + npop[0]

                cnt = lax.fori_loop(0, B3 // 16, grp, jnp.int32(0))
                zpad = jnp.zeros((16,), f32)
                for t in range(RB // 16):
                    tsl = pl.ds(cnt + t * 16, 16)
                    csrc[tsl] = jnp.zeros((16,), i32)
                    cloff[tsl] = jnp.zeros((16,), i32)
                    cp0[tsl] = zpad
                    cp1[tsl] = zpad
                    cp2[tsl] = zpad
                nblk = (cnt + RB - 1) // RB

                def flush(j, c2):
                    bo = j * RB
                    for t in range(RB // 16):
                        tsl = pl.ds(bo + t * 16, 16)
                        dsl = pl.ds(t * 16, 16)
                        sblk[dsl] = csrc[tsl]
                        oblk[dsl] = cloff[tsl]
                    pltpu.async_copy(xs_h.at[sblk], rowbuf, sem).wait()

                    def rowfn(r, c3):
                        ridx = jnp.full((16,), bo + r, i32)
                        w0 = plsc.load_gather(cp0, [ridx])
                        w1 = plsc.load_gather(cp1, [ridx])
                        w2 = plsc.load_gather(cp2, [ridx])
                        ws = (w0, w1, w2)
                        for c in range(12):
                            csl = pl.ds(c * 16, 16)
                            rowbuf[r, csl] = rowbuf[r, csl] * ws[c // 4]
                        return c3

                    lax.fori_loop(0, RB, rowfn, 0)
                    pltpu.sync_copy(rowbuf, acc_sp.at[oblk], add=True)
                    return c2

                lax.fori_loop(0, nblk, flush, 0)
                return carry

            lax.fori_loop(0, NB3, batch, 0)
            plsc.subcore_barrier()
            rows = CR // NS
            pltpu.sync_copy(acc_sp.at[pl.ds(sid * rows, rows)],
                            out_h.at[pl.ds(lo + sid * rows, rows)])
            plsc.subcore_barrier()

    return k(src, dst, p0, p1, p2, xs, zeros_chunk)


def _sc_aggregate_scalar(src, dst, p3, g3, zeros_n):
    out_type = jax.ShapeDtypeStruct((NC, NP), f32)
    scratch = [pltpu.VMEM_SHARED((NP,), f32),
               pltpu.VMEM_SHARED((NP,), f32),
               pltpu.VMEM((B,), i32),
               pltpu.VMEM((B,), i32),
               pltpu.VMEM((B,), f32),
               pltpu.VMEM((B,), f32),
               pltpu.SemaphoreType.DMA]

    @functools.partial(pl.kernel, out_type=out_type, mesh=_mesh,
                       scratch_types=scratch,
                       compiler_params=_sc_params)
    def k(src_h, dst_h, p_h, g_h, z_h, out_h,
          g_sp, acc_sp, srcv, dstv, pv, gv, sem):
        cid = lax.axis_index("c")
        sid = lax.axis_index("s")
        wid = sid * NC + cid
        ebase = wid * EW
        @pl.when(sid == 0)
        def _():
            pltpu.sync_copy(g_h, g_sp)
            pltpu.sync_copy(z_h, acc_sp)
        plsc.subcore_barrier()

        def batch(b, carry):
            off = ebase + b * B
            pltpu.sync_copy(src_h.at[pl.ds(off, B)], srcv)
            pltpu.sync_copy(dst_h.at[pl.ds(off, B)], dstv)
            pltpu.sync_copy(p_h.at[pl.ds(off, B)], pv)
            pltpu.async_copy(g_sp.at[srcv], gv, sem).wait()

            def grp(g, c2):
                sl = pl.ds(g * 16, 16)
                pv[sl] = pv[sl] * gv[sl]
                return c2

            lax.fori_loop(0, B // 16, grp, 0)
            pltpu.sync_copy(pv, acc_sp.at[dstv], add=True)
            return carry

        lax.fori_loop(0, NB, batch, 0)
        plsc.subcore_barrier()
        @pl.when(sid == 0)
        def _():
            pltpu.sync_copy(acc_sp, out_h.at[cid])
        plsc.subcore_barrier()

    return k(src, dst, p3, g3, zeros_n)


def kernel(x, edge_index, edge_attr, rand_feat, params):
    p = params
    fe = jnp.linspace(0.0, 1.0, H).astype(f32) ** 2

    u1 = fe @ p['W1c'][:H]
    u2 = fe @ p['W1c'][H:]
    ut = fe @ p['W1t']
    uc8 = jnp.zeros((8, 32), f32).at[1].set(u1).at[2].set(u2)
    ut8 = jnp.zeros((8, 32), f32).at[3].set(ut)

    ve = fe @ p['W1e']
    ve8 = jnp.zeros((8, 32), f32).at[0].set(ve)

    def edge_fold(We, ae):
        g = jnp.einsum('khc,hc->kh', We.reshape(H, 3, H), ae)
        return p['W2e'] @ g, p['b2e'] @ g

    Me1, ce1 = edge_fold(p['We1'], p['ae1'])
    Me2, ce2 = edge_fold(p['We2'], p['ae2'])
    me8 = jnp.zeros((32, 8), f32).at[:, 0:3].set(Me1).at[:, 3:6].set(Me2)
    ce8 = jnp.zeros((1, 8), f32).at[0, 0:3].set(ce1).at[0, 3:6].set(ce2)

    def proj_fold(Ws, Wd, a_s, a_d):
        Ss = jnp.stack([Ws[:, h * H:(h + 1) * H] @ a_s[h] for h in range(3)], 1)
        Sd = jnp.stack([Wd[:, h * H:(h + 1) * H] @ a_d[h] for h in range(3)], 1)
        D = Ws.shape[0]
        W = jnp.zeros((D, 256), f32)
        return W.at[:, 0:192].set(Ws).at[:, 192:195].set(Ss).at[:, 195:198].set(Sd)

    Wcat1 = proj_fold(p['Ws1'], p['Wd1'], p['as1'], p['ad1'])
    Wcat2 = proj_fold(p['Ws2'], p['Wd2'], p['as2'], p['ad2'])
    W3 = jnp.zeros((192, 128), f32)
    W3 = W3.at[:, 0].set(p['Ws3'][:, 0])
    W3 = W3.at[:, 1].set(p['Ws3'][:, 0] * p['as3'][0, 0])
    W3 = W3.at[:, 2].set(p['Wd3'][:, 0] * p['ad3'][0, 0])

    xp = jnp.zeros((NP, 8), f32).at[:N, :5].set(x)
    rfp = jnp.zeros((NP, H), f32).at[:N].set(rand_feat)
    eap = jnp.zeros((EP, 8), f32).at[:E, 0].set(edge_attr)
    src = jnp.concatenate([edge_index[0].astype(i32),
                           jnp.zeros((EP - E,), i32)])
    dst = jnp.concatenate([edge_index[1].astype(i32),
                           jnp.full((EP - E,), NP - 1, i32)])
    neginf = jnp.full((NP,), -1e30, f32)
    zeros_n = jnp.zeros((NP,), f32)
    zeros_chunk = jnp.zeros((CR, 192), f32)
    ones8 = jnp.ones((NP, 8), f32)

    xe = _tc_node_mlp(xp, uc8, ut8, p['b1c'][None], p['b1t'][None],
                      p['W2c'], p['W2t'], p['b2c'][None], p['b2t'][None])
    X1 = jnp.concatenate([xe, rfp], axis=1)
    ale = _tc_edge_terms(eap, ve8, p['b1e'][None], me8, ce8)
    ale1 = [ale[:, h] for h in range(3)]
    ale2 = [ale[:, 3 + h] for h in range(3)]

    def gat_layer(X, scale8, bias_row, Wcat, ales):
        proj = _tc_proj(X, scale8, bias_row, Wcat)
        xs = proj[:, 0:192]
        asns = [proj[:, 192 + h] for h in range(3)]
        adns = [proj[:, 195 + h] for h in range(3)]
        r1 = _sc_logits_max(3, src, dst, ales, asns, adns, neginf)
        als, mparts = r1[:3], r1[3:]
        amaxs = _tc_reduce_max(list(mparts))
        r2 = _sc_exp_den(3, dst, list(als), list(amaxs), zeros_n)
        ps, denparts = r2[:3], r2[3:]
        recips = _tc_reduce_recip(list(denparts))
        out = _sc_aggregate(src, dst, ps[0], ps[1], ps[2], xs, zeros_chunk)
        recip8 = jnp.zeros((NP, 8), f32)
        for h in range(3):
            recip8 = recip8.at[:, h].set(recips[h])
        return out, recip8

    zb128 = jnp.zeros((1, 128), f32)
    zb192 = jnp.zeros((1, 192), f32)
    out1, recip8_1 = gat_layer(X1, ones8, zb128, Wcat1, ale1)
    out2, recip8_2 = gat_layer(out1, recip8_1, p['bc1'][None], Wcat2, ale2)

    proj3 = _tc_proj(out2, recip8_2, p['bc2'][None], W3)
    g3 = proj3[:, 0]
    asn3 = [proj3[:, 1]]
    adn3 = [proj3[:, 2]]
    zl = [jnp.zeros((EP,), f32)]
    al3, mp3 = _sc_logits_max(1, src, dst, zl, asn3, adn3, neginf)
    amax3, = _tc_reduce_max([mp3])
    p3, den3 = _sc_exp_den(1, dst, [al3], [amax3], zeros_n)
    recip3, = _tc_reduce_recip([den3])
    h3part = _sc_aggregate_scalar(src, dst, p3, g3, zeros_n)
    return _tc_final(h3part, recip3, x[:, 0], p['bc3'])
